# baseline (speedup 1.0000x reference)
import functools

import jax
import jax.numpy as jnp
from jax import lax
from jax.experimental import pallas as pl
from jax.experimental.pallas import tpu as pltpu
from jax.experimental.pallas import tpu_sc as plsc

N_CODES = 1024
HIDDEN_DIM = 768
BOTTLENECK_DIM = 64
EPS = 1e-12


def _tc_body(z_ref, wzt_ref, cbt_ref, wq_ref, codes_ref, codes1d_ref,
             loss_ref, cq_ref, wz_ref, cnt_ref, csq_ref):
    i = pl.program_id(0)
    nsteps = pl.num_programs(0)

    @pl.when(i == 0)
    def _():
        loss_ref[0, 0] = 0.0
        wz_ref[...] = wzt_ref[...].T
        cb = cbt_ref[...].T
        cnorm = jnp.sqrt(jnp.sum(cb * cb, axis=1, keepdims=True))
        cn = cb / jnp.maximum(cnorm, EPS)
        cnt_ref[...] = cn.T
        csq_ref[...] = jnp.broadcast_to(jnp.sum(cn * cn, axis=1)[None, :],
                                        csq_ref.shape)
        cq_ref[...] = jnp.dot(cn, wq_ref[...], preferred_element_type=jnp.float32)

    h = jnp.dot(z_ref[...], wz_ref[...], preferred_element_type=jnp.float32)
    hnorm = jnp.sqrt(jnp.sum(h * h, axis=1, keepdims=True))
    h = h / jnp.maximum(hnorm, EPS)

    zsq = jnp.sum(h * h, axis=1, keepdims=True)
    dist = (zsq - 2.0 * jnp.dot(h, cnt_ref[...], preferred_element_type=jnp.float32)
            + csq_ref[0][None, :])

    dmin = jnp.min(dist, axis=1, keepdims=True)
    codes = jnp.argmin(dist, axis=1).astype(jnp.int32)
    codes_ref[0, 0, :] = codes
    blk = z_ref.shape[0]
    codes1d_ref[pl.ds(i * blk, blk)] = codes
    scale = 1.25 / (blk * nsteps * BOTTLENECK_DIM)
    loss_ref[0, 0] += jnp.sum(dmin) * scale


@functools.partial(jax.jit, static_argnames=("blk",))
def _run_tc(zf, W_zT, codebookT, W_q, blk=1152):
    rows = zf.shape[0]
    nblk = rows // blk
    return pl.pallas_call(
        _tc_body,
        grid=(nblk,),
        in_specs=[
            pl.BlockSpec((blk, HIDDEN_DIM), lambda i: (i, 0)),
            pl.BlockSpec((BOTTLENECK_DIM, HIDDEN_DIM), lambda i: (0, 0)),
            pl.BlockSpec((BOTTLENECK_DIM, N_CODES), lambda i: (0, 0)),
            pl.BlockSpec((BOTTLENECK_DIM, HIDDEN_DIM), lambda i: (0, 0)),
        ],
        out_specs=[
            pl.BlockSpec((1, 1, blk), lambda i: (i, 0, 0)),
            pl.BlockSpec((rows,), lambda i: (0,)),
            pl.BlockSpec(memory_space=pltpu.SMEM),
            pl.BlockSpec((N_CODES, HIDDEN_DIM), lambda i: (0, 0)),
        ],
        out_shape=[
            jax.ShapeDtypeStruct((nblk, 1, blk), jnp.int32),
            jax.ShapeDtypeStruct((rows,), jnp.int32),
            jax.ShapeDtypeStruct((1, 1), jnp.float32),
            jax.ShapeDtypeStruct((N_CODES, HIDDEN_DIM), jnp.float32),
        ],
        scratch_shapes=[
            pltpu.VMEM((HIDDEN_DIM, BOTTLENECK_DIM), jnp.float32),
            pltpu.VMEM((BOTTLENECK_DIM, N_CODES), jnp.float32),
            pltpu.VMEM((8, N_CODES), jnp.float32),
        ],
    )(zf, W_zT, codebookT, W_q)


def _onehot_body(codes_ref, cq_ref, q_ref):
    c = codes_ref[0, 0, :]
    col = jax.lax.broadcasted_iota(jnp.int32, (c.shape[0], N_CODES), 1)
    onehot = (c[:, None] == col).astype(jnp.float32)
    q_ref[...] = jnp.dot(onehot, cq_ref[...], preferred_element_type=jnp.float32)


@functools.partial(jax.jit, static_argnames=("lo_blocks",))
def _run_tc_q(codes3, cq, lo_blocks):
    nblk, _, blk = codes3.shape
    nhi = nblk - lo_blocks
    return pl.pallas_call(
        _onehot_body,
        grid=(nhi,),
        in_specs=[
            pl.BlockSpec((1, 1, blk), lambda i: (i + lo_blocks, 0, 0)),
            pl.BlockSpec((N_CODES, HIDDEN_DIM), lambda i: (0, 0)),
        ],
        out_specs=pl.BlockSpec((blk, HIDDEN_DIM), lambda i: (i, 0)),
        out_shape=jax.ShapeDtypeStruct((nhi * blk, HIDDEN_DIM), jnp.float32),
    )(codes3, cq)


def _make_sc_gather(rows):
    info = plsc.get_sparse_core_info()
    nw = info.num_cores * info.num_subcores
    b_per_w = rows // nw
    mesh = plsc.VectorSubcoreMesh(core_axis_name="c", subcore_axis_name="s")

    @functools.partial(
        pl.kernel,
        mesh=mesh,
        out_type=jax.ShapeDtypeStruct((rows, HIDDEN_DIM), jnp.float32),
        scratch_types=[
            pltpu.VMEM((b_per_w,), jnp.int32),
            pltpu.VMEM((b_per_w, HIDDEN_DIM), jnp.float32),
            pltpu.SemaphoreType.DMA,
        ],
    )
    def gather_k(table_hbm, idx_hbm, out_hbm, idx_v, buf_v, sem):
        wid = lax.axis_index("s") * info.num_cores + lax.axis_index("c")
        base = wid * b_per_w
        pltpu.sync_copy(idx_hbm.at[pl.ds(base, b_per_w)], idx_v)
        pltpu.async_copy(table_hbm.at[idx_v], buf_v, sem).wait()
        pltpu.sync_copy(buf_v, out_hbm.at[pl.ds(base, b_per_w)])

    return gather_k


_LO_BLOCKS = 2
_sc_gather = _make_sc_gather(_LO_BLOCKS * 1152)


def kernel(z, W_z, codebook, W_q):
    B, T, D = z.shape
    zf = z.reshape(B * T, D)
    codes3, codes_flat, loss, cq = _run_tc(zf, W_z.T, codebook.T, W_q)
    q_lo = _sc_gather(cq, codes_flat)
    q_hi = _run_tc_q(codes3, cq, _LO_BLOCKS)
    q = jnp.concatenate([q_lo, q_hi], axis=0)
    return (q.reshape(B, T, D), codes3.reshape(B, T), loss[0, 0])


# speedup vs baseline: 1.1467x; 1.1467x over previous
import functools

import jax
import jax.numpy as jnp
from jax import lax
from jax.experimental import pallas as pl
from jax.experimental.pallas import tpu as pltpu
from jax.experimental.pallas import tpu_sc as plsc

N_CODES = 1024
HIDDEN_DIM = 768
BOTTLENECK_DIM = 64
EPS = 1e-12


def _tc_body(z_ref, wzt_ref, cbt_ref, wq_ref, codes_ref, codes1d_ref,
             loss_ref, cq_ref, wz_ref, cnt_ref, csq_ref):
    i = pl.program_id(0)
    nsteps = pl.num_programs(0)

    @pl.when(i == 0)
    def _():
        loss_ref[0, 0] = 0.0
        wz_ref[...] = wzt_ref[...].T
        cb = cbt_ref[...].T
        cnorm = jnp.sqrt(jnp.sum(cb * cb, axis=1, keepdims=True))
        cn = cb / jnp.maximum(cnorm, EPS)
        cnt_ref[...] = cn.T
        csq_ref[...] = jnp.broadcast_to(jnp.sum(cn * cn, axis=1)[None, :],
                                        csq_ref.shape)
        cq_ref[...] = jnp.dot(cn, wq_ref[...], preferred_element_type=jnp.float32)

    h = jnp.dot(z_ref[...], wz_ref[...], preferred_element_type=jnp.float32)
    hnorm = jnp.sqrt(jnp.sum(h * h, axis=1, keepdims=True))
    h = h / jnp.maximum(hnorm, EPS)

    zsq = jnp.sum(h * h, axis=1, keepdims=True)
    dist = (zsq - 2.0 * jnp.dot(h, cnt_ref[...], preferred_element_type=jnp.float32)
            + csq_ref[0][None, :])

    dmin = jnp.min(dist, axis=1, keepdims=True)
    codes = jnp.argmin(dist, axis=1).astype(jnp.int32)
    codes_ref[0, 0, :] = codes
    blk = z_ref.shape[0]
    codes1d_ref[pl.ds(i * blk, blk)] = codes
    scale = 1.25 / (blk * nsteps * BOTTLENECK_DIM)
    loss_ref[0, 0] += jnp.sum(dmin) * scale


@functools.partial(jax.jit, static_argnames=("blk",))
def _run_tc(zf, W_zT, codebookT, W_q, blk=1152):
    rows = zf.shape[0]
    nblk = rows // blk
    return pl.pallas_call(
        _tc_body,
        grid=(nblk,),
        in_specs=[
            pl.BlockSpec((blk, HIDDEN_DIM), lambda i: (i, 0)),
            pl.BlockSpec((BOTTLENECK_DIM, HIDDEN_DIM), lambda i: (0, 0)),
            pl.BlockSpec((BOTTLENECK_DIM, N_CODES), lambda i: (0, 0)),
            pl.BlockSpec((BOTTLENECK_DIM, HIDDEN_DIM), lambda i: (0, 0)),
        ],
        out_specs=[
            pl.BlockSpec((1, 1, blk), lambda i: (i, 0, 0)),
            pl.BlockSpec((rows,), lambda i: (0,)),
            pl.BlockSpec(memory_space=pltpu.SMEM),
            pl.BlockSpec((N_CODES, HIDDEN_DIM), lambda i: (0, 0)),
        ],
        out_shape=[
            jax.ShapeDtypeStruct((nblk, 1, blk), jnp.int32),
            jax.ShapeDtypeStruct((rows,), jnp.int32),
            jax.ShapeDtypeStruct((1, 1), jnp.float32),
            jax.ShapeDtypeStruct((N_CODES, HIDDEN_DIM), jnp.float32),
        ],
        scratch_shapes=[
            pltpu.VMEM((HIDDEN_DIM, BOTTLENECK_DIM), jnp.float32),
            pltpu.VMEM((BOTTLENECK_DIM, N_CODES), jnp.float32),
            pltpu.VMEM((8, N_CODES), jnp.float32),
        ],
    )(zf, W_zT, codebookT, W_q)


def _make_sc_gather(rows):
    info = plsc.get_sparse_core_info()
    nw = info.num_cores * info.num_subcores
    b_per_w = rows // nw
    mesh = plsc.VectorSubcoreMesh(core_axis_name="c", subcore_axis_name="s")

    @functools.partial(
        pl.kernel,
        mesh=mesh,
        out_type=jax.ShapeDtypeStruct((rows, HIDDEN_DIM), jnp.float32),
        scratch_types=[
            pltpu.VMEM((b_per_w,), jnp.int32),
            pltpu.VMEM((b_per_w, HIDDEN_DIM), jnp.float32),
            pltpu.SemaphoreType.DMA,
        ],
    )
    def gather_k(table_hbm, idx_hbm, out_hbm, idx_v, buf_v, sem):
        wid = lax.axis_index("s") * info.num_cores + lax.axis_index("c")
        base = wid * b_per_w
        pltpu.sync_copy(idx_hbm.at[pl.ds(base, b_per_w)], idx_v)
        pltpu.async_copy(table_hbm.at[idx_v], buf_v, sem).wait()
        pltpu.sync_copy(buf_v, out_hbm.at[pl.ds(base, b_per_w)])

    return gather_k


_sc_gather = _make_sc_gather(8 * 576)


def kernel(z, W_z, codebook, W_q):
    B, T, D = z.shape
    zf = z.reshape(B * T, D)
    codes3, codes_flat, loss, cq = _run_tc(zf, W_z.T, codebook.T, W_q)
    q = _sc_gather(cq, codes_flat)
    return (q.reshape(B, T, D), codes3.reshape(B, T), loss[0, 0])


# speedup vs baseline: 1.1531x; 1.0056x over previous
import functools

import jax
import jax.numpy as jnp
from jax import lax
from jax.experimental import pallas as pl
from jax.experimental.pallas import tpu as pltpu
from jax.experimental.pallas import tpu_sc as plsc

N_CODES = 1024
HIDDEN_DIM = 768
BOTTLENECK_DIM = 64
EPS = 1e-12


def _tc_body(z_ref, wzt_ref, cbt_ref, wq_ref, codes_ref, codes1d_ref,
             loss_ref, cq_ref, wz_ref, cnt_ref, csq_ref):
    i = pl.program_id(0)
    nsteps = pl.num_programs(0)

    @pl.when(i == 0)
    def _():
        loss_ref[0, 0] = 0.0
        wz_ref[...] = wzt_ref[...].T
        cb = cbt_ref[...].T
        cnorm = jnp.sqrt(jnp.sum(cb * cb, axis=1, keepdims=True))
        cn = cb / jnp.maximum(cnorm, EPS)
        cnt_ref[...] = cn.T
        csq_ref[...] = jnp.broadcast_to(jnp.sum(cn * cn, axis=1)[None, :],
                                        csq_ref.shape)
        cq_ref[...] = jnp.dot(cn, wq_ref[...], preferred_element_type=jnp.float32)

    h = jnp.dot(z_ref[...], wz_ref[...], preferred_element_type=jnp.float32)
    hnorm = jnp.sqrt(jnp.sum(h * h, axis=1, keepdims=True))
    h = h / jnp.maximum(hnorm, EPS)

    zsq = jnp.sum(h * h, axis=1, keepdims=True)
    dist = (zsq - 2.0 * jnp.dot(h, cnt_ref[...], preferred_element_type=jnp.float32)
            + csq_ref[0][None, :])

    dmin = jnp.min(dist, axis=1, keepdims=True)
    codes = jnp.argmin(dist, axis=1).astype(jnp.int32)
    codes_ref[0, 0, :] = codes
    blk = z_ref.shape[0]
    codes1d_ref[pl.ds(i * blk, blk)] = codes
    scale = 1.25 / (blk * nsteps * BOTTLENECK_DIM)
    loss_ref[0, 0] += jnp.sum(dmin) * scale


@functools.partial(jax.jit, static_argnames=("blk",))
def _run_tc(zf, W_zT, codebookT, W_q, blk=768):
    rows = zf.shape[0]
    nblk = rows // blk
    return pl.pallas_call(
        _tc_body,
        grid=(nblk,),
        in_specs=[
            pl.BlockSpec((blk, HIDDEN_DIM), lambda i: (i, 0)),
            pl.BlockSpec((BOTTLENECK_DIM, HIDDEN_DIM), lambda i: (0, 0)),
            pl.BlockSpec((BOTTLENECK_DIM, N_CODES), lambda i: (0, 0)),
            pl.BlockSpec((BOTTLENECK_DIM, HIDDEN_DIM), lambda i: (0, 0)),
        ],
        out_specs=[
            pl.BlockSpec((1, 1, blk), lambda i: (i, 0, 0)),
            pl.BlockSpec((rows,), lambda i: (0,)),
            pl.BlockSpec(memory_space=pltpu.SMEM),
            pl.BlockSpec((N_CODES, HIDDEN_DIM), lambda i: (0, 0)),
        ],
        out_shape=[
            jax.ShapeDtypeStruct((nblk, 1, blk), jnp.int32),
            jax.ShapeDtypeStruct((rows,), jnp.int32),
            jax.ShapeDtypeStruct((1, 1), jnp.float32),
            jax.ShapeDtypeStruct((N_CODES, HIDDEN_DIM), jnp.float32),
        ],
        scratch_shapes=[
            pltpu.VMEM((HIDDEN_DIM, BOTTLENECK_DIM), jnp.float32),
            pltpu.VMEM((BOTTLENECK_DIM, N_CODES), jnp.float32),
            pltpu.VMEM((8, N_CODES), jnp.float32),
        ],
    )(zf, W_zT, codebookT, W_q)


def _make_sc_gather(rows):
    info = plsc.get_sparse_core_info()
    nw = info.num_cores * info.num_subcores
    b_per_w = rows // nw
    mesh = plsc.VectorSubcoreMesh(core_axis_name="c", subcore_axis_name="s")

    @functools.partial(
        pl.kernel,
        mesh=mesh,
        out_type=jax.ShapeDtypeStruct((rows, HIDDEN_DIM), jnp.float32),
        scratch_types=[
            pltpu.VMEM((b_per_w,), jnp.int32),
            pltpu.VMEM((b_per_w, HIDDEN_DIM), jnp.float32),
            pltpu.SemaphoreType.DMA,
        ],
    )
    def gather_k(table_hbm, idx_hbm, out_hbm, idx_v, buf_v, sem):
        wid = lax.axis_index("s") * info.num_cores + lax.axis_index("c")
        base = wid * b_per_w
        pltpu.sync_copy(idx_hbm.at[pl.ds(base, b_per_w)], idx_v)
        pltpu.async_copy(table_hbm.at[idx_v], buf_v, sem).wait()
        pltpu.sync_copy(buf_v, out_hbm.at[pl.ds(base, b_per_w)])

    return gather_k


_sc_gather = _make_sc_gather(8 * 576)


def kernel(z, W_z, codebook, W_q):
    B, T, D = z.shape
    zf = z.reshape(B * T, D)
    codes3, codes_flat, loss, cq = _run_tc(zf, W_z.T, codebook.T, W_q)
    q = _sc_gather(cq, codes_flat)
    return (q.reshape(B, T, D), codes3.reshape(B, T), loss[0, 0])
